# Initial kernel scaffold; baseline (speedup 1.0000x reference)
import jax
import jax.numpy as jnp
from jax.experimental import pallas as pl


def kernel(rpn_probs, rpn_bbox, anchors):
    raise NotImplementedError("write your pallas kernel here")


# speedup vs baseline: 19.9556x; 19.9556x over previous
import jax
import jax.numpy as jnp
from jax.experimental import pallas as pl
from jax.experimental.pallas import tpu as pltpu

_STD = (0.1, 0.1, 0.2, 0.2)
_PRE = 6000
_OUT = 2000
_THR = 0.7
_ROWS = 48
_LANES = 128
_PAD = _ROWS * _LANES


def _nms_body(y1a, x1a, y2a, x2a, d0, d1, d2, d3, out,
              by1, bx1, by2, bx2, area, sup):
    ha = y2a[...] - y1a[...]
    wa = x2a[...] - x1a[...]
    cy = y1a[...] + 0.5 * ha + (d0[...] * _STD[0]) * ha
    cx = x1a[...] + 0.5 * wa + (d1[...] * _STD[1]) * wa
    h = ha * jnp.exp(d2[...] * _STD[2])
    w = wa * jnp.exp(d3[...] * _STD[3])
    y1 = cy - 0.5 * h
    x1 = cx - 0.5 * w
    y2 = y1 + h
    x2 = x1 + w
    y1 = jnp.clip(y1, 0.0, 1.0)
    x1 = jnp.clip(x1, 0.0, 1.0)
    y2 = jnp.clip(y2, 0.0, 1.0)
    x2 = jnp.clip(x2, 0.0, 1.0)
    by1[...] = y1
    bx1[...] = x1
    by2[...] = y2
    bx2[...] = x2
    area[...] = (y2 - y1) * (x2 - x1)
    sup[...] = jnp.zeros((_ROWS, _LANES), jnp.float32)
    out[...] = jnp.zeros((_OUT, 4), jnp.float32)

    lane = jax.lax.broadcasted_iota(jnp.int32, (1, _LANES), 1)
    jg = (jax.lax.broadcasted_iota(jnp.int32, (_ROWS, _LANES), 0) * _LANES
          + jax.lax.broadcasted_iota(jnp.int32, (_ROWS, _LANES), 1))

    def extract(ref, r, onehot):
        row = ref[pl.ds(r, 1), :]
        return jnp.sum(jnp.where(onehot, row, 0.0))

    def body(carry):
        i, cnt = carry
        r = i // _LANES
        c = i % _LANES
        onehot = lane == c
        s_i = extract(sup, r, onehot)
        active = s_i < 0.5

        @pl.when(active)
        def _():
            y1s = extract(by1, r, onehot)
            x1s = extract(bx1, r, onehot)
            y2s = extract(by2, r, onehot)
            x2s = extract(bx2, r, onehot)
            a_s = (y2s - y1s) * (x2s - x1s)
            yy1 = jnp.maximum(y1s, by1[...])
            xx1 = jnp.maximum(x1s, bx1[...])
            yy2 = jnp.minimum(y2s, by2[...])
            xx2 = jnp.minimum(x2s, bx2[...])
            inter = jnp.maximum(yy2 - yy1, 0.0) * jnp.maximum(xx2 - xx1, 0.0)
            union = a_s + area[...] - inter
            hit = (inter > _THR * union) & (jg > i)
            sup[...] = jnp.maximum(sup[...], hit.astype(jnp.float32))
            out[pl.ds(cnt, 1), pl.ds(0, 1)] = y1s.reshape(1, 1)
            out[pl.ds(cnt, 1), pl.ds(1, 1)] = x1s.reshape(1, 1)
            out[pl.ds(cnt, 1), pl.ds(2, 1)] = y2s.reshape(1, 1)
            out[pl.ds(cnt, 1), pl.ds(3, 1)] = x2s.reshape(1, 1)

        return i + 1, cnt + active.astype(jnp.int32)

    def cond(carry):
        i, cnt = carry
        return (i < _PRE) & (cnt < _OUT)

    jax.lax.while_loop(cond, body, (jnp.int32(0), jnp.int32(0)))


def _proposals(planes):
    return pl.pallas_call(
        _nms_body,
        grid=(planes[0].shape[0],),
        in_specs=[pl.BlockSpec((None, _ROWS, _LANES), lambda b: (b, 0, 0))] * 8,
        out_specs=pl.BlockSpec((None, _OUT, 4), lambda b: (b, 0, 0)),
        scratch_shapes=[pltpu.VMEM((_ROWS, _LANES), jnp.float32)] * 6,
        out_shape=jax.ShapeDtypeStruct((planes[0].shape[0], _OUT, 4),
                                       jnp.float32),
        compiler_params=pltpu.CompilerParams(
            dimension_semantics=("arbitrary",)),
    )(*planes)


def kernel(rpn_probs, rpn_bbox, anchors):
    batch = rpn_probs.shape[0]
    scores = rpn_probs[:, :, 1]
    _, ix = jax.lax.top_k(scores, _PRE)
    deltas = jnp.take_along_axis(rpn_bbox, ix[:, :, None], axis=1)
    anc = jnp.take_along_axis(anchors, ix[:, :, None], axis=1)
    pad = ((0, 0), (0, _PAD - _PRE), (0, 0))
    anc = jnp.pad(anc, pad)
    deltas = jnp.pad(deltas, pad)
    planes = [anc[:, :, k].reshape(batch, _ROWS, _LANES) for k in range(4)]
    planes += [deltas[:, :, k].reshape(batch, _ROWS, _LANES) for k in range(4)]
    return _proposals(planes)
